# baseline (speedup 1.0000x reference)
import functools

import jax
import jax.numpy as jnp
from jax.experimental import pallas as pl

NUM_GROUPS = 2
NUM_VARS = 8192

BM = 512
BN = 1024


def _matmul_kernel(a_ref, w_ref, b_ref, out_ref):
    acc = jax.lax.dot_general(
        a_ref[...], w_ref[...],
        dimension_numbers=(((1,), (1,)), ((), ())),
        preferred_element_type=jnp.float32,
    )
    out_ref[...] = acc + b_ref[...]


@functools.partial(jax.jit, static_argnames=())
def _logits(hidden2d, W_proj, b_proj2d):
    B, H = hidden2d.shape
    O = W_proj.shape[0]
    grid = (B // BM, O // BN)
    return pl.pallas_call(
        _matmul_kernel,
        grid=grid,
        in_specs=[
            pl.BlockSpec((BM, H), lambda i, j: (i, 0)),
            pl.BlockSpec((BN, H), lambda i, j: (j, 0)),
            pl.BlockSpec((1, BN), lambda i, j: (0, j)),
        ],
        out_specs=pl.BlockSpec((BM, BN), lambda i, j: (i, j)),
        out_shape=jax.ShapeDtypeStruct((B, O), jnp.float32),
    )(hidden2d, W_proj, b_proj2d)


def kernel(hidden_states, W_proj, b_proj, embeddings):
    batch_size = hidden_states.shape[0]
    hidden2d = hidden_states.reshape(batch_size, -1)
    logits = _logits(hidden2d, W_proj, b_proj.reshape(1, -1))
    logits = logits.reshape(batch_size, NUM_GROUPS, NUM_VARS)
    indices = jnp.argmax(logits, axis=-1)
    one_hot = jax.nn.one_hot(indices, NUM_VARS, dtype=jnp.float32)
    marginal_probs = one_hot.mean(axis=0)
    perplexity = jnp.exp(-jnp.sum(marginal_probs * jnp.log(marginal_probs + 1e-07), axis=-1)).sum()
    selected_embeddings = jnp.einsum('bgv,gve->bge', one_hot, embeddings)
    return (selected_embeddings, perplexity)


# speedup vs baseline: 4.5010x; 4.5010x over previous
import jax
import jax.numpy as jnp
from jax import lax
from jax.experimental import pallas as pl
from jax.experimental.pallas import tpu as pltpu
from jax.experimental.pallas import tpu_sc as plsc

NUM_GROUPS = 2
NUM_VARS = 8192
CODEVECTOR_DIM = 256
HIDDEN = 1024
BATCH = 4096

BM = 512
BN = 2048
J = (NUM_GROUPS * NUM_VARS) // BN
I = BATCH // BM
SPG = NUM_VARS // BN

_GW = 128


def _proj_argmax_kernel(a_ref, w_ref, b_ref, idx_ref, perp_ref,
                        rmax_ref, rcur_ref, ridx0_ref, counts_ref):
    j = pl.program_id(0)
    i = pl.program_id(1)

    @pl.when(jnp.logical_and(j == 0, i == 0))
    def _init():
        counts_ref[...] = jnp.zeros_like(counts_ref)

    acc = lax.dot_general(
        a_ref[...], w_ref[...],
        dimension_numbers=(((1,), (1,)), ((), ())),
        preferred_element_type=jnp.float32,
    ) + b_ref[...]

    lmax = jnp.max(acc, axis=1, keepdims=True)
    lane = lax.broadcasted_iota(jnp.int32, (BM, BN), 1)
    lidx = jnp.min(jnp.where(acc == lmax, lane, BN), axis=1, keepdims=True)
    lidx = lidx + j * BN

    rows = pl.ds(i * BM, BM)
    jg = lax.rem(j, SPG)

    @pl.when(jg == 0)
    def _start_group():
        rmax_ref[rows, :] = lmax
        rcur_ref[rows, :] = lidx

    @pl.when(jg != 0)
    def _update():
        prev = rmax_ref[rows, :]
        upd = lmax > prev
        rmax_ref[rows, :] = jnp.where(upd, lmax, prev)
        rcur_ref[rows, :] = jnp.where(upd, lidx, rcur_ref[rows, :])

    def _histogram(g, win):
        v = win - g * NUM_VARS
        bins = lax.broadcasted_iota(jnp.int32, (1, NUM_VARS), 1)
        hits = jnp.sum((v == bins).astype(jnp.float32), axis=0,
                       keepdims=True)
        counts_ref[g:g + 1, :] += hits

    @pl.when(j == SPG - 1)
    def _end_g0():
        win = rcur_ref[rows, :]
        ridx0_ref[rows, :] = win
        _histogram(0, win)

    @pl.when(j == J - 1)
    def _end_g1():
        win = rcur_ref[rows, :]
        idx_ref[:, 0:1] = ridx0_ref[rows, :]
        idx_ref[:, 1:2] = win
        _histogram(1, win)

    @pl.when(jnp.logical_and(j == J - 1, i == I - 1))
    def _perplexity():
        p = counts_ref[...] * (1.0 / BATCH)
        s = jnp.sum(p * jnp.log(p + 1e-7), axis=1, keepdims=True)
        perp_ref[...] = jnp.sum(jnp.exp(-s), axis=0, keepdims=True)


def _proj_argmax(hidden2d, W_proj, b_proj2d):
    return pl.pallas_call(
        _proj_argmax_kernel,
        grid=(J, I),
        in_specs=[
            pl.BlockSpec((BM, HIDDEN), lambda j, i: (i, 0)),
            pl.BlockSpec((BN, HIDDEN), lambda j, i: (j, 0)),
            pl.BlockSpec((1, BN), lambda j, i: (0, j)),
        ],
        out_specs=[
            pl.BlockSpec((BM, NUM_GROUPS), lambda j, i: (i, 0)),
            pl.BlockSpec((1, 1), lambda j, i: (0, 0)),
        ],
        out_shape=[
            jax.ShapeDtypeStruct((BATCH, NUM_GROUPS), jnp.int32),
            jax.ShapeDtypeStruct((1, 1), jnp.float32),
        ],
        scratch_shapes=[
            pltpu.VMEM((BATCH, 1), jnp.float32),
            pltpu.VMEM((BATCH, 1), jnp.int32),
            pltpu.VMEM((BATCH, 1), jnp.int32),
            pltpu.VMEM((NUM_GROUPS, NUM_VARS), jnp.float32),
        ],
    )(hidden2d, W_proj, b_proj2d)


def _sc_gather(table, flat_idx):
    n = flat_idx.shape[1]

    @pl.kernel(
        out_type=jax.ShapeDtypeStruct((n, CODEVECTOR_DIM), table.dtype),
        mesh=plsc.VectorSubcoreMesh(core_axis_name="core",
                                    subcore_axis_name="subcore"),
    )
    def gather_kernel(tab_hbm, idx_hbm, out_hbm):
        def body(i_vmem, o_vmem):
            pltpu.sync_copy(tab_hbm.at[i_vmem.at[0]], o_vmem)

        pltpu.emit_pipeline(
            body,
            grid=(n // _GW,),
            in_specs=[pl.BlockSpec((1, _GW), lambda i: (0, i))],
            out_specs=[pl.BlockSpec((_GW, CODEVECTOR_DIM), lambda i: (i, 0))],
            core_axis_name=("core", "subcore"),
            dimension_semantics=(pltpu.PARALLEL,),
        )(idx_hbm, out_hbm)

    return gather_kernel(table, flat_idx)


def kernel(hidden_states, W_proj, b_proj, embeddings):
    batch = hidden_states.shape[0]
    hidden2d = hidden_states.reshape(batch, HIDDEN)
    idx, perp = _proj_argmax(hidden2d, W_proj, b_proj.reshape(1, -1))
    table = embeddings.reshape(NUM_GROUPS * NUM_VARS, CODEVECTOR_DIM)
    gathered = _sc_gather(table, idx.reshape(1, batch * NUM_GROUPS))
    selected = gathered.reshape(batch, NUM_GROUPS, CODEVECTOR_DIM)
    return (selected, perp.reshape(()))
